# Initial kernel scaffold; baseline (speedup 1.0000x reference)
import jax
import jax.numpy as jnp
from jax.experimental import pallas as pl


def kernel(x, table):
    raise NotImplementedError("write your pallas kernel here")


# speedup vs baseline: 4.8061x; 4.8061x over previous
import functools

import jax
import jax.numpy as jnp
from jax import lax
from jax.experimental import pallas as pl
from jax.experimental.pallas import tpu as pltpu
from jax.experimental.pallas import tpu_sc as plsc

DIM = 32
NUM_CORES = 2
NUM_SUBCORES = 16
NW = NUM_CORES * NUM_SUBCORES

CHUNK = 1024


def _sc_gather(idx_flat, table, b_total):
    b_per_w = b_total // NW
    n_chunks = b_per_w // CHUNK
    mesh = plsc.VectorSubcoreMesh(core_axis_name="c", subcore_axis_name="s")

    @functools.partial(
        pl.kernel,
        mesh=mesh,
        out_type=jax.ShapeDtypeStruct((b_total, DIM), jnp.float32),
        scratch_types=[
            pltpu.VMEM((CHUNK,), jnp.int32),
            pltpu.VMEM((CHUNK, DIM), jnp.float32),
            pltpu.SemaphoreType.DMA,
        ],
        compiler_params=pltpu.CompilerParams(use_tc_tiling_on_sc=False),
    )
    def k(idx_hbm, table_hbm, out_hbm, idx_v, rows_v, sem):
        wid = lax.axis_index("s") * NUM_CORES + lax.axis_index("c")
        base = wid * b_per_w

        def body(i, carry):
            off = base + i * CHUNK
            pltpu.sync_copy(idx_hbm.at[pl.ds(off, CHUNK)], idx_v)
            pltpu.async_copy(table_hbm.at[idx_v], rows_v, sem).wait()
            pltpu.sync_copy(rows_v, out_hbm.at[pl.ds(off, CHUNK)])
            return carry

        lax.fori_loop(0, n_chunks, body, 0)

    return k(idx_flat, table)


def kernel(x, table):
    b_total = x.shape[0] * x.shape[1]
    idx_flat = x.reshape(b_total).astype(jnp.int32)
    out = _sc_gather(idx_flat, table, b_total)
    return out.reshape(x.shape[0], x.shape[1], DIM)


# speedup vs baseline: 5.0475x; 1.0502x over previous
import functools

import jax
import jax.numpy as jnp
from jax import lax
from jax.experimental import pallas as pl
from jax.experimental.pallas import tpu as pltpu
from jax.experimental.pallas import tpu_sc as plsc

DIM = 32
NUM_CORES = 2
NUM_SUBCORES = 16
NW = NUM_CORES * NUM_SUBCORES

CHUNK = 1024
NBUF = 2


def _sc_gather(idx_flat, table, b_total):
    b_per_w = b_total // NW
    n_chunks = b_per_w // CHUNK
    mesh = plsc.VectorSubcoreMesh(core_axis_name="c", subcore_axis_name="s")

    @functools.partial(
        pl.kernel,
        mesh=mesh,
        out_type=jax.ShapeDtypeStruct((b_total, DIM), jnp.float32),
        scratch_types=[
            pltpu.VMEM((NBUF, CHUNK), jnp.int32),
            pltpu.VMEM((NBUF, CHUNK, DIM), jnp.float32),
            pltpu.SemaphoreType.DMA((NBUF,)),
            pltpu.SemaphoreType.DMA((NBUF,)),
            pltpu.SemaphoreType.DMA((NBUF,)),
        ],
        compiler_params=pltpu.CompilerParams(use_tc_tiling_on_sc=False),
    )
    def k(idx_hbm, table_hbm, out_hbm, idx_b, rows_b, sem_idx, sem_g, sem_o):
        wid = lax.axis_index("s") * NUM_CORES + lax.axis_index("c")
        base = wid * b_per_w

        def idx_copy(i, s):
            off = base + i * CHUNK
            return pltpu.make_async_copy(
                idx_hbm.at[pl.ds(off, CHUNK)], idx_b.at[s], sem_idx.at[s])

        def gather(s):
            return pltpu.make_async_copy(
                table_hbm.at[idx_b.at[s]], rows_b.at[s], sem_g.at[s])

        def store(i, s):
            off = base + i * CHUNK
            return pltpu.make_async_copy(
                rows_b.at[s], out_hbm.at[pl.ds(off, CHUNK)], sem_o.at[s])

        for s in range(NBUF):
            idx_copy(s, s).start()
        idx_copy(0, 0).wait()
        gather(0).start()

        @pl.loop(0, n_chunks, step=NBUF)
        def group(g):
            for s in range(NBUF):
                i = g + s
                s1 = (s + 1) % NBUF

                @pl.when(i + 1 < n_chunks)
                def _():
                    idx_copy(i + 1, s1).wait()

                    @pl.when(i + 1 >= NBUF)
                    def _():
                        store(i + 1 - NBUF, s1).wait()

                    gather(s1).start()

                gather(s).wait()
                store(i, s).start()

                @pl.when(i + NBUF < n_chunks)
                def _():
                    idx_copy(i + NBUF, s).start()

        for s in range(NBUF):
            i = n_chunks - NBUF + s
            store(i, s % NBUF).wait()

    return k(idx_flat, table)


def kernel(x, table):
    b_total = x.shape[0] * x.shape[1]
    idx_flat = x.reshape(b_total).astype(jnp.int32)
    out = _sc_gather(idx_flat, table, b_total)
    return out.reshape(x.shape[0], x.shape[1], DIM)
